# baseline (speedup 1.0000x reference)
import functools

import jax
import jax.numpy as jnp
from jax import lax
from jax.experimental import pallas as pl
from jax.experimental.pallas import tpu as pltpu
from jax.experimental.pallas import tpu_sc as plsc

NC = 2
NS = 16
NW = NC * NS
LANES = 16
EPS = 4
LW = 256
NBUF = 2


def _tree_sum(vals):
    while len(vals) > 1:
        nxt = [vals[i] + vals[i + 1] for i in range(0, len(vals) - 1, 2)]
        if len(vals) % 2:
            nxt.append(vals[-1])
        vals = nxt
    return vals[0]


def _make_pool_kernel(B, L, D, V):
    NR = B // EPS
    SPW = NR // NW
    BPW = B // NW
    D2 = 2 * D
    mesh = plsc.VectorSubcoreMesh(core_axis_name="c", subcore_axis_name="s")

    @functools.partial(
        pl.kernel,
        mesh=mesh,
        out_type=jax.ShapeDtypeStruct((B, D), jnp.float32),
        scratch_types=[
            pltpu.VMEM((SPW * LW,), jnp.int32),
            [pltpu.VMEM((LW, D2), jnp.float32)] * NBUF,
            [pltpu.VMEM((EPS * L * LANES,), jnp.float32)] * NBUF,
            pltpu.VMEM((BPW, D), jnp.float32),
            [pltpu.SemaphoreType.DMA] * NBUF,
            [pltpu.SemaphoreType.DMA] * NBUF,
        ],
    )
    def pool(kidx_hbm, hws_hbm, table_hbm, out_hbm,
             idx_v, bufs, wbufs, out_v, sems, wsems):
        WPR = EPS * L * LANES
        wid = lax.axis_index("s") * NC + lax.axis_index("c")
        base = wid * SPW
        pltpu.sync_copy(kidx_hbm.at[pl.ds(base * LW, SPW * LW)], idx_v)

        def issue(s, k):
            pltpu.async_copy(
                table_hbm.at[idx_v.at[pl.ds(s * LW, LW)]], bufs[k], sems[k]
            )
            pltpu.async_copy(
                hws_hbm.at[pl.ds((base + s) * WPR, WPR)], wbufs[k], wsems[k]
            )

        for k in range(NBUF):
            issue(k, k)

        def process(rows, hws, s):
            @pl.loop(0, EPS)
            def _(e):
                accs = [None] * (D // LANES)
                for r in range(L):
                    w = hws[pl.ds((e * L + r) * LANES, LANES)]
                    for j in range(D // LANES):
                        a = rows[e * L + r, pl.ds(j * LANES, LANES)]
                        b = rows[e * L + r, pl.ds(D + j * LANES, LANES)]
                        v = a + (b - a) * w
                        accs[j] = v if accs[j] is None else accs[j] + v
                for j in range(D // LANES):
                    out_v[EPS * s + e, pl.ds(j * LANES, LANES)] = accs[j]

        @pl.loop(0, SPW, step=NBUF)
        def _(s):
            for k in range(NBUF):
                pltpu.make_async_copy(
                    table_hbm.at[idx_v.at[pl.ds(0, LW)]], bufs[k], sems[k]
                ).wait()
                pltpu.make_async_copy(
                    hws_hbm.at[pl.ds(0, WPR)], wbufs[k], wsems[k]
                ).wait()
                process(bufs[k], wbufs[k], s + k)

                @pl.when(s + NBUF + k < SPW)
                def _():
                    issue(s + NBUF + k, k)

        pltpu.sync_copy(out_v, out_hbm.at[pl.ds(wid * BPW, BPW)])

    return pool


def _project(pooled, Ws, b2d):
    B, D = pooled.shape
    T = Ws.shape[1]

    def body(x_ref, w_ref, b_ref, o_ref):
        o_ref[...] = (
            jnp.dot(x_ref[...], w_ref[...], preferred_element_type=jnp.float32)
            + b_ref[...]
        )

    return pl.pallas_call(
        body,
        out_shape=jax.ShapeDtypeStruct((B, T), jnp.float32),
    )(pooled, Ws, b2d)


@jax.jit
def kernel(text_ids, table, W, b):
    B, L = text_ids.shape
    V, D = table.shape
    T = W.shape[1]
    ids = text_ids.astype(jnp.int32).reshape(B // EPS, EPS * L)
    kidx = jnp.pad(ids >> 1, ((0, 0), (0, LW - EPS * L))).reshape(-1)
    hws = jnp.broadcast_to(
        (ids & 1).astype(jnp.float32)[:, :, None],
        (B // EPS, EPS * L, LANES),
    ).reshape(-1)
    table2 = table.reshape(V // 2, 2 * D)
    pooled = _make_pool_kernel(B, L, D, V)(kidx, hws, table2)
    Ws = W * (1.0 / L)
    return _project(pooled, Ws, b.reshape(1, T))


# speedup vs baseline: 3.3911x; 3.3911x over previous
import functools

import jax
import jax.numpy as jnp
from jax import lax
from jax.experimental import pallas as pl
from jax.experimental.pallas import tpu as pltpu
from jax.experimental.pallas import tpu_sc as plsc

NC = 2
NS = 16
NW = NC * NS
LANES = 16
EPR = 2
RPS = 4
NBUF = 4


def _tree_sum(vals):
    while len(vals) > 1:
        nxt = [vals[i] + vals[i + 1] for i in range(0, len(vals) - 1, 2)]
        if len(vals) % 2:
            nxt.append(vals[-1])
        vals = nxt
    return vals[0]


def _make_pool_kernel(B, L, D, V):
    LW = EPR * RPS * L
    NR = B // (EPR * RPS)
    SPW = NR // NW
    EPS = EPR * RPS
    BPW = B // NW
    mesh = plsc.VectorSubcoreMesh(core_axis_name="c", subcore_axis_name="s")

    @functools.partial(
        pl.kernel,
        mesh=mesh,
        compiler_params=pltpu.CompilerParams(
            use_tc_tiling_on_sc=False, needs_layout_passes=False
        ),
        out_type=jax.ShapeDtypeStruct((B, D), jnp.float32),
        scratch_types=[
            pltpu.VMEM((SPW, LW), jnp.int32),
            [pltpu.VMEM((LW, D), jnp.bfloat16)] * NBUF,
            pltpu.VMEM((BPW, D), jnp.float32),
            [pltpu.SemaphoreType.DMA] * NBUF,
        ],
    )
    def pool(ids_hbm, table_hbm, out_hbm, idx_v, bufs, out_v, sems):
        wid = lax.axis_index("s") * NC + lax.axis_index("c")
        base = wid * SPW
        pltpu.sync_copy(ids_hbm.at[pl.ds(base, SPW)], idx_v)

        def issue(s, k):
            pltpu.async_copy(table_hbm.at[idx_v.at[s]], bufs[k], sems[k])

        for k in range(NBUF):
            issue(k, k)

        def process(rows, s):
            @pl.loop(0, EPS)
            def _(e):
                for q in range(D // 32):
                    evens, odds = [], []
                    for r in range(L):
                        x = plsc.bitcast(
                            rows[e * L + r, pl.ds(32 * q, 32)], jnp.int32
                        )
                        evens.append(plsc.bitcast(x << 16, jnp.float32))
                        odds.append(
                            plsc.bitcast(x & jnp.int32(-65536), jnp.float32)
                        )
                    out_v[EPS * s + e, pl.ds(32 * q, LANES)] = _tree_sum(evens)
                    out_v[EPS * s + e, pl.ds(32 * q + LANES, LANES)] = (
                        _tree_sum(odds)
                    )

        @pl.loop(0, SPW, step=NBUF)
        def _(s):
            for k in range(NBUF):
                pltpu.make_async_copy(
                    table_hbm.at[idx_v.at[0]], bufs[k], sems[k]
                ).wait()
                process(bufs[k], s + k)

                @pl.when(s + NBUF + k < SPW)
                def _():
                    issue(s + NBUF + k, k)

        pltpu.sync_copy(out_v, out_hbm.at[pl.ds(wid * BPW, BPW)])

    return pool


def _project(pooled, Ws, b2d):
    B, D = pooled.shape
    T = Ws.shape[1]

    def body(x_ref, w_ref, b_ref, o_ref):
        o_ref[...] = (
            jnp.dot(x_ref[...], w_ref[...], preferred_element_type=jnp.float32)
            + b_ref[...]
        )

    return pl.pallas_call(
        body,
        out_shape=jax.ShapeDtypeStruct((B, T), jnp.float32),
    )(pooled, Ws, b2d)


@jax.jit
def kernel(text_ids, table, W, b):
    B, L = text_ids.shape
    V, D = table.shape
    T = W.shape[1]
    ids2 = text_ids.astype(jnp.int32).reshape(B // (EPR * RPS), EPR * RPS * L)
    pooled = _make_pool_kernel(B, L, D, V)(ids2, table.astype(jnp.bfloat16))
    perm = []
    for q in range(D // 32):
        perm += [32 * q + 2 * k for k in range(16)]
        perm += [32 * q + 2 * k + 1 for k in range(16)]
    Ws = W[jnp.asarray(perm)] * (1.0 / L)
    return _project(pooled, Ws, b.reshape(1, T))


# speedup vs baseline: 4.4996x; 1.3269x over previous
import functools

import jax
import jax.numpy as jnp
from jax import lax
from jax.experimental import pallas as pl
from jax.experimental.pallas import tpu as pltpu
from jax.experimental.pallas import tpu_sc as plsc

NC = 2
NS = 16
NW = NC * NS
LANES = 16
EPR = 2
RPS = 4
NBUF = 4


def _tree_sum(vals):
    while len(vals) > 1:
        nxt = [vals[i] + vals[i + 1] for i in range(0, len(vals) - 1, 2)]
        if len(vals) % 2:
            nxt.append(vals[-1])
        vals = nxt
    return vals[0]


def _make_pool_kernel(B, L, D, V):
    LW = EPR * RPS * L
    NR = B // (EPR * RPS)
    SPW = NR // NW
    EPS = EPR * RPS
    BPW = B // NW
    mesh = plsc.VectorSubcoreMesh(core_axis_name="c", subcore_axis_name="s")

    @functools.partial(
        pl.kernel,
        mesh=mesh,
        compiler_params=pltpu.CompilerParams(use_tc_tiling_on_sc=False),
        out_type=jax.ShapeDtypeStruct((B, D), jnp.float32),
        scratch_types=[
            pltpu.VMEM((SPW, LW), jnp.int32),
            [pltpu.VMEM((LW, D), jnp.float32)] * NBUF,
            pltpu.VMEM((BPW, D), jnp.float32),
            [pltpu.SemaphoreType.DMA] * NBUF,
        ],
    )
    def pool(ids_hbm, table_hbm, out_hbm, idx_v, bufs, out_v, sems):
        wid = lax.axis_index("s") * NC + lax.axis_index("c")
        base = wid * SPW
        pltpu.sync_copy(ids_hbm.at[pl.ds(base, SPW)], idx_v)

        def issue(s, k):
            pltpu.async_copy(table_hbm.at[idx_v.at[s]], bufs[k], sems[k])

        for k in range(NBUF):
            issue(k, k)

        def process(rows, s):
            @pl.loop(0, EPS)
            def _(e):
                for j in range(D // LANES):
                    sl = pl.ds(j * LANES, LANES)
                    out_v[EPS * s + e, sl] = _tree_sum(
                        [rows[e * L + r, sl] for r in range(L)]
                    )

        @pl.loop(0, SPW, step=NBUF)
        def _(s):
            for k in range(NBUF):
                pltpu.make_async_copy(
                    table_hbm.at[idx_v.at[0]], bufs[k], sems[k]
                ).wait()
                process(bufs[k], s + k)

                @pl.when(s + NBUF + k < SPW)
                def _():
                    issue(s + NBUF + k, k)

        pltpu.sync_copy(out_v, out_hbm.at[pl.ds(wid * BPW, BPW)])

    return pool


def _project(pooled, Ws, b2d):
    B, D = pooled.shape
    T = Ws.shape[1]

    def body(x_ref, w_ref, b_ref, o_ref):
        o_ref[...] = (
            jnp.dot(x_ref[...], w_ref[...], preferred_element_type=jnp.float32)
            + b_ref[...]
        )

    return pl.pallas_call(
        body,
        out_shape=jax.ShapeDtypeStruct((B, T), jnp.float32),
    )(pooled, Ws, b2d)


@jax.jit
def kernel(text_ids, table, W, b):
    B, L = text_ids.shape
    V, D = table.shape
    T = W.shape[1]
    ids2 = text_ids.astype(jnp.int32).reshape(B // (EPR * RPS), EPR * RPS * L)
    pooled = _make_pool_kernel(B, L, D, V)(ids2, table)
    Ws = W * (1.0 / L)
    return _project(pooled, Ws, b.reshape(1, T))
